# baseline (speedup 1.0000x reference)
import functools

import jax
import jax.numpy as jnp
from jax import lax
from jax.experimental import pallas as pl
from jax.experimental.pallas import tpu as pltpu
from jax.experimental.pallas import tpu_sc as plsc

N_NODES = 10000
N_PAD = 10240
TRASH = 10016
N_EDGES = 320000
D = 128
NC = 2
NS = 16
CHUNK = 128
NCHUNK_TOTAL = 2560
E_PAD = NCHUNK_TOTAL * CHUNK
NCHUNK = NCHUNK_TOTAL // (NC * NS)
IBLK = 4
SUPER = NCHUNK // (2 * IBLK)
R_BAND = N_PAD // NS


def _layer_body(p0_ref, p1_ref, x_ref, b_ref, w_ref, out_ref):
    agg = p0_ref[0] + p1_ref[0] - x_ref[...]
    out_ref[...] = jnp.tanh(
        jnp.dot(agg, w_ref[...], preferred_element_type=jnp.float32)
        + b_ref[...])


def _layer(p, x, b, w, blk=2000):
    return pl.pallas_call(
        _layer_body,
        grid=(N_NODES // blk,),
        in_specs=[pl.BlockSpec((1, blk, D), lambda i: (0, i, 0)),
                  pl.BlockSpec((1, blk, D), lambda i: (1, i, 0)),
                  pl.BlockSpec((blk, D), lambda i: (i, 0)),
                  pl.BlockSpec((1, D), lambda i: (0, 0)),
                  pl.BlockSpec((D, D), lambda i: (0, 0))],
        out_specs=pl.BlockSpec((blk, D), lambda i: (i, 0)),
        out_shape=jax.ShapeDtypeStruct((N_PAD, D), jnp.float32),
    )(p, p, x, b.reshape(1, D), w)


def _final_body(q0_ref, q1_ref, h1_ref, b_ref, w_ref, wo_ref, bo_ref,
                out_ref, acc):
    i = pl.program_id(0)
    agg = q0_ref[0] + q1_ref[0] - h1_ref[...]
    h2 = jnp.tanh(
        jnp.dot(agg, w_ref[...], preferred_element_type=jnp.float32)
        + b_ref[...])
    node = jnp.dot(h2, wo_ref[...], preferred_element_type=jnp.float32)
    s = jnp.sum(node, axis=0, keepdims=True)

    @pl.when(i == 0)
    def _():
        acc[...] = jnp.broadcast_to(s, acc.shape)

    @pl.when(i > 0)
    def _():
        acc[...] += jnp.broadcast_to(s, acc.shape)

    @pl.when(i == pl.num_programs(0) - 1)
    def _():
        out_ref[...] = jnp.tanh(acc[...][0:1, :] * (1.0 / N_NODES)
                                + bo_ref[...])


def _final_readout(q, h1, b, w, wo_pad, bo_pad, blk=2000):
    one = pl.BlockSpec((1, D), lambda i: (0, 0))
    full = pl.BlockSpec((D, D), lambda i: (0, 0))
    return pl.pallas_call(
        _final_body,
        grid=(N_NODES // blk,),
        in_specs=[pl.BlockSpec((1, blk, D), lambda i: (0, i, 0)),
                  pl.BlockSpec((1, blk, D), lambda i: (1, i, 0)),
                  pl.BlockSpec((blk, D), lambda i: (i, 0)),
                  one, full, full, one],
        out_specs=one,
        out_shape=jax.ShapeDtypeStruct((1, D), jnp.float32),
        scratch_shapes=[pltpu.VMEM((8, D), jnp.float32)],
    )(q, q, h1, b.reshape(1, D), w, wo_pad, bo_pad)


def _sc_agg(y, ei):
    mesh = plsc.VectorSubcoreMesh(core_axis_name="c", subcore_axis_name="s")

    @functools.partial(
        pl.kernel,
        out_type=jax.ShapeDtypeStruct((NC, N_PAD, D), jnp.float32),
        mesh=mesh,
        scratch_types=[
            pltpu.VMEM((IBLK, 2, CHUNK), jnp.int32),
            pltpu.VMEM((IBLK, 2, CHUNK), jnp.int32),
            pltpu.VMEM((CHUNK, D), jnp.float32),
            pltpu.VMEM((CHUNK, D), jnp.float32),
            pltpu.VMEM_SHARED((N_PAD, D), jnp.float32),
            pltpu.SemaphoreType.DMA,
            pltpu.SemaphoreType.DMA,
            pltpu.SemaphoreType.DMA,
            pltpu.SemaphoreType.DMA,
        ],
    )
    def k(y_hbm, ei_hbm, p_hbm, ib0, ib1, buf_a, buf_b, acc_sh,
          sem_i0, sem_i1, sem_a, sem_b):
        c = lax.axis_index("c")
        s = lax.axis_index("s")
        r0 = s * R_BAND
        base = (c * NS + s) * NCHUNK

        pltpu.sync_copy(y_hbm.at[pl.ds(r0, R_BAND)],
                        acc_sh.at[pl.ds(r0, R_BAND)])
        plsc.subcore_barrier()

        pltpu.sync_copy(ei_hbm.at[pl.ds(base, IBLK)], ib0)
        pltpu.async_copy(ei_hbm.at[pl.ds(base + IBLK, IBLK)], ib1, sem_i1)
        pltpu.async_copy(y_hbm.at[ib0.at[0, 0]], buf_a, sem_a)

        def body(m, carry):
            blk0 = base + 2 * m * IBLK

            for p in range(2 * IBLK):
                ib, prow = (ib0, p) if p < IBLK else (ib1, p - IBLK)
                rbuf, rsem = (buf_a, sem_a) if p % 2 == 0 else (buf_b, sem_b)
                nbuf, nsem = (buf_b, sem_b) if p % 2 == 0 else (buf_a, sem_a)

                if p == 0:
                    @pl.when(m > 0)
                    def _():
                        pltpu.async_copy(
                            ei_hbm.at[pl.ds(blk0 + IBLK, IBLK)], ib1, sem_i1)

                pltpu.make_async_copy(y_hbm.at[ib.at[prow, 0]],
                                      rbuf, rsem).wait()

                if p == IBLK:
                    @pl.when(m < SUPER - 1)
                    def _():
                        pltpu.async_copy(
                            ei_hbm.at[pl.ds(blk0 + 2 * IBLK, IBLK)],
                            ib0, sem_i0)

                if p < IBLK - 1:
                    pltpu.async_copy(y_hbm.at[ib0.at[prow + 1, 0]],
                                     nbuf, nsem)
                elif p == IBLK - 1:
                    pltpu.make_async_copy(
                        ei_hbm.at[pl.ds(blk0 + IBLK, IBLK)],
                        ib1, sem_i1).wait()
                    pltpu.async_copy(y_hbm.at[ib1.at[0, 0]], nbuf, nsem)
                elif p < 2 * IBLK - 1:
                    pltpu.async_copy(y_hbm.at[ib1.at[prow + 1, 0]],
                                     nbuf, nsem)
                else:
                    @pl.when(m < SUPER - 1)
                    def _():
                        pltpu.make_async_copy(
                            ei_hbm.at[pl.ds(blk0 + 2 * IBLK, IBLK)],
                            ib0, sem_i0).wait()
                        pltpu.async_copy(y_hbm.at[ib0.at[0, 0]], nbuf, nsem)

                pltpu.sync_copy(rbuf, acc_sh.at[ib.at[prow, 1]], add=True)
            return carry

        lax.fori_loop(0, SUPER, body, 0)
        plsc.subcore_barrier()
        pltpu.sync_copy(acc_sh.at[pl.ds(r0, R_BAND)],
                        p_hbm.at[c, pl.ds(r0, R_BAND)])

    return k(y, ei)


def kernel(x, edge_index, pos, W1, b1, W2, b2, Wout, bout):
    del pos
    ei = edge_index.astype(jnp.int32)
    ei = jnp.concatenate(
        [ei, jnp.full((2, E_PAD - N_EDGES), TRASH, jnp.int32)], axis=1)
    ei3 = ei.reshape(2, NCHUNK_TOTAL, CHUNK).transpose(1, 0, 2)

    x_pad = jnp.pad(x, ((0, N_PAD - N_NODES), (0, 0)))
    p = _sc_agg(x_pad, ei3)
    h1 = _layer(p, x_pad, b1, W1)
    q = _sc_agg(h1, ei3)

    wo_pad = jnp.zeros((D, D), jnp.float32).at[:, :2].set(Wout)
    bo_pad = jnp.zeros((1, D), jnp.float32).at[0, :2].set(bout)
    out_pad = _final_readout(q, h1, b2, W2, wo_pad, bo_pad)
    return out_pad[0, :2]


# speedup vs baseline: 3.2794x; 3.2794x over previous
import functools

import jax
import jax.numpy as jnp
from jax import lax
from jax.experimental import pallas as pl
from jax.experimental.pallas import tpu as pltpu
from jax.experimental.pallas import tpu_sc as plsc

N_NODES = 10000
N_PAD = 10240
TRASH = 10016
N_EDGES = 320000
D = 128
NC = 2
NS = 16
CHUNK = 128
NCHUNK_TOTAL = 2560
E_PAD = NCHUNK_TOTAL * CHUNK
NCHUNK = NCHUNK_TOTAL // (NC * NS)
IBLK = 4
SUPER = NCHUNK // (2 * IBLK)
R_BAND = N_PAD // NS


def _layer_body(p0_ref, p1_ref, x_ref, b_ref, w_ref, out_ref):
    agg = p0_ref[0] + p1_ref[0] - x_ref[...]
    out_ref[...] = jnp.tanh(
        jnp.dot(agg, w_ref[...], preferred_element_type=jnp.float32)
        + b_ref[...])


def _layer(p, x, b, w, blk=2000):
    return pl.pallas_call(
        _layer_body,
        grid=(N_NODES // blk,),
        in_specs=[pl.BlockSpec((1, blk, D), lambda i: (0, i, 0)),
                  pl.BlockSpec((1, blk, D), lambda i: (1, i, 0)),
                  pl.BlockSpec((blk, D), lambda i: (i, 0)),
                  pl.BlockSpec((1, D), lambda i: (0, 0)),
                  pl.BlockSpec((D, D), lambda i: (0, 0))],
        out_specs=pl.BlockSpec((blk, D), lambda i: (i, 0)),
        out_shape=jax.ShapeDtypeStruct((N_PAD, D), jnp.float32),
    )(p, p, x, b.reshape(1, D), w)


def _final_body(q0_ref, q1_ref, h1_ref, b_ref, w_ref, wo_ref, bo_ref,
                out_ref, acc):
    i = pl.program_id(0)
    agg = q0_ref[0] + q1_ref[0] - h1_ref[...]
    h2 = jnp.tanh(
        jnp.dot(agg, w_ref[...], preferred_element_type=jnp.float32)
        + b_ref[...])
    node = jnp.dot(h2, wo_ref[...], preferred_element_type=jnp.float32)
    s = jnp.sum(node, axis=0, keepdims=True)

    @pl.when(i == 0)
    def _():
        acc[...] = jnp.broadcast_to(s, acc.shape)

    @pl.when(i > 0)
    def _():
        acc[...] += jnp.broadcast_to(s, acc.shape)

    @pl.when(i == pl.num_programs(0) - 1)
    def _():
        out_ref[...] = jnp.tanh(acc[...][0:1, :] * (1.0 / N_NODES)
                                + bo_ref[...])


def _final_readout(q, h1, b, w, wo_pad, bo_pad, blk=2000):
    one = pl.BlockSpec((1, D), lambda i: (0, 0))
    full = pl.BlockSpec((D, D), lambda i: (0, 0))
    return pl.pallas_call(
        _final_body,
        grid=(N_NODES // blk,),
        in_specs=[pl.BlockSpec((1, blk, D), lambda i: (0, i, 0)),
                  pl.BlockSpec((1, blk, D), lambda i: (1, i, 0)),
                  pl.BlockSpec((blk, D), lambda i: (i, 0)),
                  one, full, full, one],
        out_specs=one,
        out_shape=jax.ShapeDtypeStruct((1, D), jnp.float32),
        scratch_shapes=[pltpu.VMEM((8, D), jnp.float32)],
    )(q, q, h1, b.reshape(1, D), w, wo_pad, bo_pad)


def _sc_agg(y, ei):
    mesh = plsc.VectorSubcoreMesh(core_axis_name="c", subcore_axis_name="s")

    @functools.partial(
        pl.kernel,
        out_type=jax.ShapeDtypeStruct((NC, N_PAD, D), jnp.float32),
        mesh=mesh,
        scratch_types=[
            pltpu.VMEM((IBLK, 2, CHUNK), jnp.int32),
            pltpu.VMEM((IBLK, 2, CHUNK), jnp.int32),
            pltpu.VMEM((CHUNK, D), jnp.float32),
            pltpu.VMEM((CHUNK, D), jnp.float32),
            pltpu.VMEM_SHARED((N_PAD, D), jnp.float32),
            pltpu.SemaphoreType.DMA,
            pltpu.SemaphoreType.DMA,
            pltpu.SemaphoreType.DMA,
            pltpu.SemaphoreType.DMA,
        ],
    )
    def k(y_hbm, ei_hbm, p_hbm, ib0, ib1, buf_a, buf_b, acc_sh,
          sem_i0, sem_i1, sem_a, sem_b):
        c = lax.axis_index("c")
        s = lax.axis_index("s")
        r0 = s * R_BAND
        base = (c * NS + s) * NCHUNK

        pltpu.sync_copy(y_hbm.at[pl.ds(r0, R_BAND)],
                        acc_sh.at[pl.ds(r0, R_BAND)])
        plsc.subcore_barrier()

        pltpu.sync_copy(ei_hbm.at[pl.ds(base, IBLK)], ib0)
        pltpu.async_copy(ei_hbm.at[pl.ds(base + IBLK, IBLK)], ib1, sem_i1)
        pltpu.async_copy(y_hbm.at[ib0.at[0, 0]], buf_a, sem_a)

        def body(m, carry):
            blk0 = base + 2 * m * IBLK

            for p in range(2 * IBLK):
                ib, prow = (ib0, p) if p < IBLK else (ib1, p - IBLK)
                rbuf, rsem = (buf_a, sem_a) if p % 2 == 0 else (buf_b, sem_b)
                nbuf, nsem = (buf_b, sem_b) if p % 2 == 0 else (buf_a, sem_a)

                if p == 0:
                    @pl.when(m > 0)
                    def _():
                        pltpu.async_copy(
                            ei_hbm.at[pl.ds(blk0 + IBLK, IBLK)], ib1, sem_i1)

                pltpu.make_async_copy(y_hbm.at[ib.at[prow, 0]],
                                      rbuf, rsem).wait()

                if p == IBLK:
                    @pl.when(m < SUPER - 1)
                    def _():
                        pltpu.async_copy(
                            ei_hbm.at[pl.ds(blk0 + 2 * IBLK, IBLK)],
                            ib0, sem_i0)

                if p < IBLK - 1:
                    pltpu.async_copy(y_hbm.at[ib0.at[prow + 1, 0]],
                                     nbuf, nsem)
                elif p == IBLK - 1:
                    pltpu.make_async_copy(
                        ei_hbm.at[pl.ds(blk0 + IBLK, IBLK)],
                        ib1, sem_i1).wait()
                    pltpu.async_copy(y_hbm.at[ib1.at[0, 0]], nbuf, nsem)
                elif p < 2 * IBLK - 1:
                    pltpu.async_copy(y_hbm.at[ib1.at[prow + 1, 0]],
                                     nbuf, nsem)
                else:
                    @pl.when(m < SUPER - 1)
                    def _():
                        pltpu.make_async_copy(
                            ei_hbm.at[pl.ds(blk0 + 2 * IBLK, IBLK)],
                            ib0, sem_i0).wait()
                        pltpu.async_copy(y_hbm.at[ib0.at[0, 0]], nbuf, nsem)

                pltpu.sync_copy(rbuf, acc_sh.at[ib.at[prow, 1]], add=True)
            return carry

        lax.fori_loop(0, SUPER, body, 0)
        plsc.subcore_barrier()
        pltpu.sync_copy(acc_sh.at[pl.ds(r0, R_BAND)],
                        p_hbm.at[c, pl.ds(r0, R_BAND)])

    return k(y, ei)


def kernel(x, edge_index, pos, W1, b1, W2, b2, Wout, bout):
    del pos
    ei = edge_index.astype(jnp.int32)
    fill = N_NODES + jnp.arange(E_PAD - N_EDGES, dtype=jnp.int32) % (
        N_PAD - N_NODES)
    ei = jnp.concatenate([ei, jnp.stack([fill, fill])], axis=1)
    ei3 = ei.reshape(2, NCHUNK_TOTAL, CHUNK).transpose(1, 0, 2)

    x_pad = jnp.pad(x, ((0, N_PAD - N_NODES), (0, 0)))
    p = _sc_agg(x_pad, ei3)
    h1 = _layer(p, x_pad, b1, W1)
    q = _sc_agg(h1, ei3)

    wo_pad = jnp.zeros((D, D), jnp.float32).at[:, :2].set(Wout)
    bo_pad = jnp.zeros((1, D), jnp.float32).at[0, :2].set(bout)
    out_pad = _final_readout(q, h1, b2, W2, wo_pad, bo_pad)
    return out_pad[0, :2]
